# baseline (speedup 1.0000x reference)
import functools

import jax
import jax.numpy as jnp
from jax.experimental import pallas as pl
from jax.experimental.pallas import tpu as pltpu

_BN = 256


def _fused_gc_body(d_ref, Tm_ref, adj_ref, HW_ref, b_ref, out_ref, *, bn, num_k, relu):
    k = pl.program_id(1)
    Tk = Tm_ref[pl.ds(k * bn, bn), :] * d_ref[...]
    mult = jax.lax.dot_general(
        Tm_ref[...], Tk, (((1,), (1,)), ((), ())),
        preferred_element_type=jnp.float32,
    )
    adj = adj_ref[...]
    bm = adj.shape[0]
    i = pl.program_id(0)
    rows = jax.lax.broadcasted_iota(jnp.int32, (bm, bn), 0) + i * bm
    cols = jax.lax.broadcasted_iota(jnp.int32, (bm, bn), 1) + k * bn
    A = jnp.where(rows == cols, adj, mult * adj)
    HWk = HW_ref[pl.ds(k * bn, bn), :]
    contrib = jax.lax.dot_general(
        A, HWk, (((1,), (0,)), ((), ())),
        preferred_element_type=jnp.float32,
    )

    @pl.when(k == 0)
    def _():
        out_ref[...] = b_ref[...] + contrib

    @pl.when(k > 0)
    def _():
        out_ref[...] += contrib

    if relu:
        @pl.when(k == num_k - 1)
        def _():
            out_ref[...] = jnp.maximum(out_ref[...], 0.0)


def _fused_gc(Tm, d, adj, HW, b, relu):
    R, K = Tm.shape
    F = HW.shape[1]
    bn = _BN
    num_k = R // bn
    ni = 2
    bm = R // ni
    return pl.pallas_call(
        functools.partial(_fused_gc_body, bn=bn, num_k=num_k, relu=relu),
        grid=(ni, num_k),
        in_specs=[
            pl.BlockSpec((1, K), lambda i, k: (0, 0)),
            pl.BlockSpec((bm, K), lambda i, k: (i, 0)),
            pl.BlockSpec((bm, bn), lambda i, k: (i, k)),
            pl.BlockSpec((R, F), lambda i, k: (0, 0)),
            pl.BlockSpec((1, F), lambda i, k: (0, 0)),
        ],
        out_specs=pl.BlockSpec((bm, F), lambda i, k: (i, 0)),
        out_shape=jax.ShapeDtypeStruct((R, F), jnp.float32),
        compiler_params=pltpu.CompilerParams(
            dimension_semantics=("parallel", "arbitrary"),
            vmem_limit_bytes=56 * 1024 * 1024,
        ),
    )(d.reshape(1, K), Tm, adj, HW, b.reshape(1, F))


def _layernorm(h, g, be):
    m = jnp.mean(h, axis=-1, keepdims=True)
    v = jnp.mean((h - m) ** 2, axis=-1, keepdims=True)
    return (h - m) / jnp.sqrt(v + 1e-5) * g + be


def _prep1_body(X_ref, Z_ref, W_ref, p_ref, fW_ref, g_ref, be_ref,
                d_ref, HW_ref, F1_ref):
    X = X_ref[...]
    Z = Z_ref[...]
    d_ref[...] = jax.lax.dot_general(
        p_ref[...], Z, (((1,), (1,)), ((), ())),
        preferred_element_type=jnp.float32)
    HW_ref[...] = jnp.dot(X, W_ref[...], preferred_element_type=jnp.float32)
    h = jnp.dot(X, fW_ref[...], preferred_element_type=jnp.float32)
    F1_ref[...] = jnp.maximum(_layernorm(h, g_ref[...], be_ref[...]), 0.0)


def _prep1(X, Z, W, p, fW, g, be):
    N_, NFV_ = X.shape
    E_ = Z.shape[0]
    NH = W.shape[1]
    return pl.pallas_call(
        _prep1_body,
        out_shape=(
            jax.ShapeDtypeStruct((1, E_), jnp.float32),
            jax.ShapeDtypeStruct((N_, NH), jnp.float32),
            jax.ShapeDtypeStruct((N_, NH), jnp.float32),
        ),
    )(X, Z, W, p, fW, g.reshape(1, NH), be.reshape(1, NH))


def _prep2_body(X1F1_ref, Z_ref, W_ref, p_ref, fW_ref, g_ref, be_ref,
                d_ref, HeW_ref, F2_ref):
    Z = Z_ref[...]
    d_ref[...] = jax.lax.dot_general(
        p_ref[...], X1F1_ref[...], (((1,), (1,)), ((), ())),
        preferred_element_type=jnp.float32)
    Z1 = jnp.maximum(Z, 0.0)
    HeW_ref[...] = jnp.dot(Z1, W_ref[...], preferred_element_type=jnp.float32)
    h = jnp.dot(Z, fW_ref[...], preferred_element_type=jnp.float32)
    F2_ref[...] = jnp.maximum(_layernorm(h, g_ref[...], be_ref[...]), 0.0)


def _prep2(X1F1, Z, W, p, fW, g, be):
    N_ = X1F1.shape[0]
    E_, NFE_ = Z.shape
    return pl.pallas_call(
        _prep2_body,
        out_shape=(
            jax.ShapeDtypeStruct((1, N_), jnp.float32),
            jax.ShapeDtypeStruct((E_, NFE_), jnp.float32),
            jax.ShapeDtypeStruct((E_, NFE_), jnp.float32),
        ),
    )(X1F1, Z, W, p, fW, g.reshape(1, NFE_), be.reshape(1, NFE_))


def _prep35_body(Hv_ref, He_ref, W_ref, p_ref, d_ref, HW_ref):
    d_ref[...] = jax.lax.dot_general(
        p_ref[...], He_ref[...], (((1,), (1,)), ((), ())),
        preferred_element_type=jnp.float32)
    HW_ref[...] = jnp.dot(Hv_ref[...], W_ref[...],
                          preferred_element_type=jnp.float32)


def _prep35(Hv, He, W, p):
    return pl.pallas_call(
        _prep35_body,
        out_shape=(
            jax.ShapeDtypeStruct((1, He.shape[0]), jnp.float32),
            jax.ShapeDtypeStruct((Hv.shape[0], W.shape[1]), jnp.float32),
        ),
    )(Hv, He, W, p)


def _prep4_body(Hv_ref, He_ref, W_ref, p_ref, d_ref, HeW_ref):
    d_ref[...] = jax.lax.dot_general(
        p_ref[...], Hv_ref[...], (((1,), (1,)), ((), ())),
        preferred_element_type=jnp.float32)
    HeW_ref[...] = jnp.dot(He_ref[...], W_ref[...],
                           preferred_element_type=jnp.float32)


def _prep4(Hv, He, W, p):
    return pl.pallas_call(
        _prep4_body,
        out_shape=(
            jax.ShapeDtypeStruct((1, Hv.shape[0]), jnp.float32),
            jax.ShapeDtypeStruct((He.shape[0], W.shape[1]), jnp.float32),
        ),
    )(Hv, He, W, p)


def kernel(X, Z, adj_e, adj_v, T, gc1_W, gc1_p, gc1_b, fc1_W, fc1_g, fc1_be,
           gc2_W, gc2_p, gc2_b, fc2_W, fc2_g, fc2_be, gc3_W, gc3_p, gc3_b,
           gc4_W, gc4_p, gc4_b, gc5_W, gc5_p, gc5_b):
    Tb = T
    Ttb = T.T

    d1, HW1, F1 = _prep1(X, Z, gc1_W, gc1_p, fc1_W, fc1_g, fc1_be)
    X1 = _fused_gc(Tb, d1, adj_v, HW1, gc1_b, relu=True)
    X1F1 = jnp.concatenate([X1, F1], axis=1)

    d2, HeW2, F2 = _prep2(X1F1, Z, gc2_W, gc2_p, fc2_W, fc2_g, fc2_be)
    Z2 = _fused_gc(Ttb, d2, adj_e, HeW2, gc2_b, relu=True)
    Z2F2 = jnp.concatenate([Z2, F2], axis=1)

    d3, HW3 = _prep35(X1F1, Z2F2, gc3_W, gc3_p)
    X3 = _fused_gc(Tb, d3, adj_v, HW3, gc3_b, relu=True)

    d4, HeW4 = _prep4(X3, Z2F2, gc4_W, gc4_p)
    Z4 = _fused_gc(Ttb, d4, adj_e, HeW4, gc4_b, relu=True)

    d5, HW5 = _prep35(X3, Z4, gc5_W, gc5_p)
    X5 = _fused_gc(Tb, d5, adj_v, HW5, gc5_b, relu=False)
    return X5


# speedup vs baseline: 1.0237x; 1.0237x over previous
import functools

import jax
import jax.numpy as jnp
from jax.experimental import pallas as pl
from jax.experimental.pallas import tpu as pltpu

_BN = 256


def _fused_gc_body(d_ref, Tm_ref, adj_ref, HW_ref, b_ref, out_ref, *, bn, num_k, relu):
    k = pl.program_id(1)
    Tk = Tm_ref[pl.ds(k * bn, bn), :] * d_ref[...]
    mult = jax.lax.dot_general(
        Tm_ref[...], Tk, (((1,), (1,)), ((), ())),
        preferred_element_type=jnp.float32,
    )
    adj = adj_ref[...]
    bm = adj.shape[0]
    i = pl.program_id(0)
    rows = jax.lax.broadcasted_iota(jnp.int32, (bm, bn), 0) + i * bm
    cols = jax.lax.broadcasted_iota(jnp.int32, (bm, bn), 1) + k * bn
    A = jnp.where(rows == cols, adj, mult * adj)
    HWk = HW_ref[pl.ds(k * bn, bn), :]
    contrib = jax.lax.dot_general(
        A, HWk, (((1,), (0,)), ((), ())),
        preferred_element_type=jnp.float32,
    )

    @pl.when(k == 0)
    def _():
        out_ref[...] = b_ref[...] + contrib

    @pl.when(k > 0)
    def _():
        out_ref[...] += contrib

    if relu:
        @pl.when(k == num_k - 1)
        def _():
            out_ref[...] = jnp.maximum(out_ref[...], 0.0)


def _fused_gc(Tm, d, adj, HW, b, relu):
    R, K = Tm.shape
    F = HW.shape[1]
    bn = _BN
    num_k = R // bn
    ni = 1
    bm = R // ni
    return pl.pallas_call(
        functools.partial(_fused_gc_body, bn=bn, num_k=num_k, relu=relu),
        grid=(ni, num_k),
        in_specs=[
            pl.BlockSpec((1, K), lambda i, k: (0, 0)),
            pl.BlockSpec((bm, K), lambda i, k: (i, 0)),
            pl.BlockSpec((bm, bn), lambda i, k: (i, k)),
            pl.BlockSpec((R, F), lambda i, k: (0, 0)),
            pl.BlockSpec((1, F), lambda i, k: (0, 0)),
        ],
        out_specs=pl.BlockSpec((bm, F), lambda i, k: (i, 0)),
        out_shape=jax.ShapeDtypeStruct((R, F), jnp.float32),
        compiler_params=pltpu.CompilerParams(
            dimension_semantics=("parallel", "arbitrary"),
            vmem_limit_bytes=56 * 1024 * 1024,
        ),
    )(d.reshape(1, K), Tm, adj, HW, b.reshape(1, F))


def _layernorm(h, g, be):
    m = jnp.mean(h, axis=-1, keepdims=True)
    v = jnp.mean((h - m) ** 2, axis=-1, keepdims=True)
    return (h - m) / jnp.sqrt(v + 1e-5) * g + be


def _prep1_body(X_ref, Z_ref, W_ref, p_ref, fW_ref, g_ref, be_ref,
                d_ref, HW_ref, F1_ref):
    X = X_ref[...]
    Z = Z_ref[...]
    d_ref[...] = jax.lax.dot_general(
        p_ref[...], Z, (((1,), (1,)), ((), ())),
        preferred_element_type=jnp.float32)
    HW_ref[...] = jnp.dot(X, W_ref[...], preferred_element_type=jnp.float32)
    h = jnp.dot(X, fW_ref[...], preferred_element_type=jnp.float32)
    F1_ref[...] = jnp.maximum(_layernorm(h, g_ref[...], be_ref[...]), 0.0)


def _prep1(X, Z, W, p, fW, g, be):
    N_, NFV_ = X.shape
    E_ = Z.shape[0]
    NH = W.shape[1]
    return pl.pallas_call(
        _prep1_body,
        out_shape=(
            jax.ShapeDtypeStruct((1, E_), jnp.float32),
            jax.ShapeDtypeStruct((N_, NH), jnp.float32),
            jax.ShapeDtypeStruct((N_, NH), jnp.float32),
        ),
    )(X, Z, W, p, fW, g.reshape(1, NH), be.reshape(1, NH))


def _prep2_body(X1F1_ref, Z_ref, W_ref, p_ref, fW_ref, g_ref, be_ref,
                d_ref, HeW_ref, F2_ref):
    Z = Z_ref[...]
    d_ref[...] = jax.lax.dot_general(
        p_ref[...], X1F1_ref[...], (((1,), (1,)), ((), ())),
        preferred_element_type=jnp.float32)
    Z1 = jnp.maximum(Z, 0.0)
    HeW_ref[...] = jnp.dot(Z1, W_ref[...], preferred_element_type=jnp.float32)
    h = jnp.dot(Z, fW_ref[...], preferred_element_type=jnp.float32)
    F2_ref[...] = jnp.maximum(_layernorm(h, g_ref[...], be_ref[...]), 0.0)


def _prep2(X1F1, Z, W, p, fW, g, be):
    N_ = X1F1.shape[0]
    E_, NFE_ = Z.shape
    return pl.pallas_call(
        _prep2_body,
        out_shape=(
            jax.ShapeDtypeStruct((1, N_), jnp.float32),
            jax.ShapeDtypeStruct((E_, NFE_), jnp.float32),
            jax.ShapeDtypeStruct((E_, NFE_), jnp.float32),
        ),
    )(X1F1, Z, W, p, fW, g.reshape(1, NFE_), be.reshape(1, NFE_))


def _prep35_body(Hv_ref, He_ref, W_ref, p_ref, d_ref, HW_ref):
    d_ref[...] = jax.lax.dot_general(
        p_ref[...], He_ref[...], (((1,), (1,)), ((), ())),
        preferred_element_type=jnp.float32)
    HW_ref[...] = jnp.dot(Hv_ref[...], W_ref[...],
                          preferred_element_type=jnp.float32)


def _prep35(Hv, He, W, p):
    return pl.pallas_call(
        _prep35_body,
        out_shape=(
            jax.ShapeDtypeStruct((1, He.shape[0]), jnp.float32),
            jax.ShapeDtypeStruct((Hv.shape[0], W.shape[1]), jnp.float32),
        ),
    )(Hv, He, W, p)


def _prep4_body(Hv_ref, He_ref, W_ref, p_ref, d_ref, HeW_ref):
    d_ref[...] = jax.lax.dot_general(
        p_ref[...], Hv_ref[...], (((1,), (1,)), ((), ())),
        preferred_element_type=jnp.float32)
    HeW_ref[...] = jnp.dot(He_ref[...], W_ref[...],
                           preferred_element_type=jnp.float32)


def _prep4(Hv, He, W, p):
    return pl.pallas_call(
        _prep4_body,
        out_shape=(
            jax.ShapeDtypeStruct((1, Hv.shape[0]), jnp.float32),
            jax.ShapeDtypeStruct((He.shape[0], W.shape[1]), jnp.float32),
        ),
    )(Hv, He, W, p)


def kernel(X, Z, adj_e, adj_v, T, gc1_W, gc1_p, gc1_b, fc1_W, fc1_g, fc1_be,
           gc2_W, gc2_p, gc2_b, fc2_W, fc2_g, fc2_be, gc3_W, gc3_p, gc3_b,
           gc4_W, gc4_p, gc4_b, gc5_W, gc5_p, gc5_b):
    Tb = T
    Ttb = T.T

    d1, HW1, F1 = _prep1(X, Z, gc1_W, gc1_p, fc1_W, fc1_g, fc1_be)
    X1 = _fused_gc(Tb, d1, adj_v, HW1, gc1_b, relu=True)
    X1F1 = jnp.concatenate([X1, F1], axis=1)

    d2, HeW2, F2 = _prep2(X1F1, Z, gc2_W, gc2_p, fc2_W, fc2_g, fc2_be)
    Z2 = _fused_gc(Ttb, d2, adj_e, HeW2, gc2_b, relu=True)
    Z2F2 = jnp.concatenate([Z2, F2], axis=1)

    d3, HW3 = _prep35(X1F1, Z2F2, gc3_W, gc3_p)
    X3 = _fused_gc(Tb, d3, adj_v, HW3, gc3_b, relu=True)

    d4, HeW4 = _prep4(X3, Z2F2, gc4_W, gc4_p)
    Z4 = _fused_gc(Ttb, d4, adj_e, HeW4, gc4_b, relu=True)

    d5, HW5 = _prep35(X3, Z4, gc5_W, gc5_p)
    X5 = _fused_gc(Tb, d5, adj_v, HW5, gc5_b, relu=False)
    return X5
